# baseline (speedup 1.0000x reference)
import jax
import jax.numpy as jnp
from jax import lax
from jax.experimental import pallas as pl
from jax.experimental.pallas import tpu as pltpu
from jax.experimental.pallas import tpu_sc as plsc

N_NODES = 10000
N_EDGES = 160000
N_PAIRS = 20000
IN_FEATS = 256
HIDDEN = 512

NC = 2
NS = 16
LANE = 128

EDGE_B = 125
EDGE_NB = (N_EDGES // NS) // EDGE_B
EDGE_W = 16

PAD_N = 10240
ROWS_PER_TILE = PAD_N // NS

PAIR_TOT = 2 * N_PAIRS
PAIR_TOTP = 40960
PAIR_PER_TILE = PAIR_TOTP // (NC * NS)
PAIR_B = 128
PAIR_NB = PAIR_PER_TILE // PAIR_B

def _sc_mesh():
    return plsc.VectorSubcoreMesh(
        core_axis_name="c", subcore_axis_name="s",
        num_cores=NC, num_subcores=NS)


def _make_sc_agg(C):
    def body(h_hbm, srcr_hbm, dstr_hbm, zacc_hbm, agg_hbm,
             src_w, dst_w, rows_a, rows_b, acc_sh, gsem_a, gsem_b):
        cid = lax.axis_index("c")
        sid = lax.axis_index("s")
        r0 = sid * ROWS_PER_TILE

        for i in range(C // NC):
            cc = cid * (C // NC) + i
            pltpu.sync_copy(zacc_hbm.at[pl.ds(r0, ROWS_PER_TILE)],
                            acc_sh.at[pl.ds(r0, ROWS_PER_TILE)])
            plsc.subcore_barrier()

            def fire(j, buf, sem):
                pltpu.async_copy(
                    h_hbm.at[cc].at[src_w.at[j]], buf, sem)

            def drain_scat(j, buf, sem):
                pltpu.make_async_copy(
                    h_hbm.at[cc].at[src_w.at[j]], buf, sem).wait()
                pltpu.sync_copy(buf, acc_sh.at[dst_w.at[j]], add=True)

            def win_body(w, carry):
                pltpu.sync_copy(
                    srcr_hbm.at[sid, pl.ds(w * EDGE_W, EDGE_W)], src_w)
                pltpu.sync_copy(
                    dstr_hbm.at[sid, pl.ds(w * EDGE_W, EDGE_W)], dst_w)
                fire(0, rows_a, gsem_a)

                def pair_body(t, c2):
                    j0 = 2 * t
                    fire(j0 + 1, rows_b, gsem_b)
                    drain_scat(j0, rows_a, gsem_a)

                    @pl.when(j0 + 2 < EDGE_W)
                    def _():
                        fire(j0 + 2, rows_a, gsem_a)
                    drain_scat(j0 + 1, rows_b, gsem_b)
                    return c2
                lax.fori_loop(0, EDGE_W // 2, pair_body, 0)
                return carry
            lax.fori_loop(0, EDGE_NB // EDGE_W, win_body, 0)
            plsc.subcore_barrier()

            pltpu.sync_copy(acc_sh.at[pl.ds(r0, ROWS_PER_TILE)],
                            agg_hbm.at[cc, pl.ds(r0, ROWS_PER_TILE)])
            plsc.subcore_barrier()

    return pl.kernel(
        body,
        out_type=jax.ShapeDtypeStruct((C, PAD_N, LANE), jnp.float32),
        mesh=_sc_mesh(),
        scratch_types=[
            pltpu.VMEM((EDGE_W, EDGE_B), jnp.int32),
            pltpu.VMEM((EDGE_W, EDGE_B), jnp.int32),
            pltpu.VMEM((EDGE_B, LANE), jnp.float32),
            pltpu.VMEM((EDGE_B, LANE), jnp.float32),
            pltpu.VMEM_SHARED((PAD_N, LANE), jnp.float32),
            pltpu.SemaphoreType.DMA,
            pltpu.SemaphoreType.DMA,
        ],
    )


CNT_NB = N_EDGES // (NC * NS) // EDGE_B


def _sc_counts(d0r, d1r, d2r, zacc, ones):
    def body(d0_hbm, d1_hbm, d2_hbm, zacc_hbm, ones_hbm,
             c0_hbm, c1_hbm, c2_hbm, dst_v, ones_v, cnt_sh, csem):
        cid = lax.axis_index("c")
        sid = lax.axis_index("s")
        wid = cid * NS + sid
        r0 = sid * ROWS_PER_TILE
        pltpu.sync_copy(ones_hbm, ones_v)
        for d_hbm, c_hbm in ((d0_hbm, c0_hbm),
                             (d1_hbm, c1_hbm),
                             (d2_hbm, c2_hbm)):
            pltpu.sync_copy(d_hbm.at[wid], dst_v)
            pltpu.sync_copy(zacc_hbm.at[pl.ds(r0, ROWS_PER_TILE)],
                            cnt_sh.at[pl.ds(r0, ROWS_PER_TILE)])
            plsc.subcore_barrier()

            def cnt_body(j, carry):
                pltpu.async_copy(ones_v, cnt_sh.at[dst_v.at[j]],
                                 csem, add=True)
                return carry
            lax.fori_loop(0, CNT_NB, cnt_body, 0)

            def cnt_drain(j, carry):
                pltpu.make_async_copy(ones_v, cnt_sh.at[dst_v.at[j]],
                                      csem).wait()
                return carry
            lax.fori_loop(0, CNT_NB, cnt_drain, 0)
            plsc.subcore_barrier()
            pltpu.sync_copy(cnt_sh.at[pl.ds(r0, ROWS_PER_TILE)],
                            c_hbm.at[cid, pl.ds(r0, ROWS_PER_TILE)])
            plsc.subcore_barrier()

    k = pl.kernel(
        body,
        out_type=(
            jax.ShapeDtypeStruct((NC, PAD_N, LANE), jnp.float32),
            jax.ShapeDtypeStruct((NC, PAD_N, LANE), jnp.float32),
            jax.ShapeDtypeStruct((NC, PAD_N, LANE), jnp.float32),
        ),
        mesh=_sc_mesh(),
        scratch_types=[
            pltpu.VMEM((CNT_NB, EDGE_B), jnp.int32),
            pltpu.VMEM((EDGE_B, LANE), jnp.float32),
            pltpu.VMEM_SHARED((PAD_N, LANE), jnp.float32),
            pltpu.SemaphoreType.DMA,
        ],
    )
    return k(d0r, d1r, d2r, zacc, ones)


def _sc_pair_gather(h, qsrc, qdst, nb):
    per_tile = nb * PAIR_B
    tot = (NC * NS) * per_tile

    def body(h_hbm, qs_hbm, qd_hbm, ga_hbm, gb_hbm,
             qs_v, qd_v, buf_a, buf_b, sem_a, sem_b):
        cid = lax.axis_index("c")
        sid = lax.axis_index("s")
        wid = cid * NS + sid
        base = wid * per_tile
        pltpu.sync_copy(qs_hbm.at[wid], qs_v)
        pltpu.sync_copy(qd_hbm.at[wid], qd_v)
        for c in range(4):
            def pair_body(j, carry):
                cpa = pltpu.async_copy(
                    h_hbm.at[c].at[qs_v.at[j]], buf_a, sem_a)
                cpb = pltpu.async_copy(
                    h_hbm.at[c].at[qd_v.at[j]], buf_b, sem_b)
                cpa.wait()
                pltpu.sync_copy(buf_a,
                                ga_hbm.at[c, pl.ds(base + j * PAIR_B, PAIR_B)])
                cpb.wait()
                pltpu.sync_copy(buf_b,
                                gb_hbm.at[c, pl.ds(base + j * PAIR_B, PAIR_B)])
                return carry
            lax.fori_loop(0, nb, pair_body, 0)

    k = pl.kernel(
        body,
        out_type=(
            jax.ShapeDtypeStruct((4, tot, LANE), jnp.float32),
            jax.ShapeDtypeStruct((4, tot, LANE), jnp.float32),
        ),
        mesh=_sc_mesh(),
        scratch_types=[
            pltpu.VMEM((nb, PAIR_B), jnp.int32),
            pltpu.VMEM((nb, PAIR_B), jnp.int32),
            pltpu.VMEM((PAIR_B, LANE), jnp.float32),
            pltpu.VMEM((PAIR_B, LANE), jnp.float32),
            pltpu.SemaphoreType.DMA,
            pltpu.SemaphoreType.DMA,
        ],
    )
    return k(h, qsrc, qdst)


def _tc_layer(h, agg, cnt, ws, wn, b, relu):
    C = h.shape[0]
    BM = 1000
    grid = (N_NODES // BM,)

    def body(h_ref, agg_ref, cnt_ref, ws_ref, wn_ref, b_ref, out_ref):
        cnt = cnt_ref[0] + cnt_ref[1]
        inv = 1.0 / jnp.maximum(cnt, 1.0)
        s = jnp.zeros((BM, HIDDEN), jnp.float32)
        for c in range(C):
            s += jnp.dot(h_ref[c], ws_ref[c],
                         preferred_element_type=jnp.float32)
            s += jnp.dot(agg_ref[c] * inv, wn_ref[c],
                         preferred_element_type=jnp.float32)
        s += b_ref[...]
        if relu:
            s = jnp.maximum(s, 0.0)
        for c2 in range(HIDDEN // LANE):
            out_ref[c2] = s[:, c2 * LANE:(c2 + 1) * LANE]

    return pl.pallas_call(
        body,
        grid=grid,
        in_specs=[
            pl.BlockSpec((C, BM, LANE), lambda i: (0, i, 0)),
            pl.BlockSpec((C, BM, LANE), lambda i: (0, i, 0)),
            pl.BlockSpec((NC, BM, 1), lambda i: (0, i, 0)),
            pl.BlockSpec((C, LANE, HIDDEN), lambda i: (0, 0, 0)),
            pl.BlockSpec((C, LANE, HIDDEN), lambda i: (0, 0, 0)),
            pl.BlockSpec((1, HIDDEN), lambda i: (0, 0)),
        ],
        out_specs=pl.BlockSpec((HIDDEN // LANE, BM, LANE), lambda i: (0, i, 0)),
        out_shape=jax.ShapeDtypeStruct((HIDDEN // LANE, N_NODES, LANE),
                                       jnp.float32),
    )(h, agg, cnt, ws, wn, b)


def _tc_decoder(ga, gb, w1, b1, w2, b2, w3, b3):
    BM = 1024
    grid = (ga.shape[1] // BM,)

    def body(ga_ref, gb_ref, w1_ref, b1_ref, w2_ref, b2_ref, w3_ref, b3_ref,
             out_ref):
        t = jnp.zeros((BM, HIDDEN), jnp.float32)
        for c in range(4):
            e = ga_ref[c] * gb_ref[c]
            t += jnp.dot(e, w1_ref[c], preferred_element_type=jnp.float32)
        t = jnp.maximum(t + b1_ref[...], 0.0)
        t = jnp.maximum(
            jnp.dot(t, w2_ref[...], preferred_element_type=jnp.float32)
            + b2_ref[...], 0.0)
        out_ref[...] = (
            jnp.dot(t, w3_ref[...], preferred_element_type=jnp.float32)
            + b3_ref[...])

    return pl.pallas_call(
        body,
        grid=grid,
        in_specs=[
            pl.BlockSpec((4, BM, LANE), lambda i: (0, i, 0)),
            pl.BlockSpec((4, BM, LANE), lambda i: (0, i, 0)),
            pl.BlockSpec((4, LANE, HIDDEN), lambda i: (0, 0, 0)),
            pl.BlockSpec((1, HIDDEN), lambda i: (0, 0)),
            pl.BlockSpec((HIDDEN, HIDDEN), lambda i: (0, 0)),
            pl.BlockSpec((1, HIDDEN), lambda i: (0, 0)),
            pl.BlockSpec((HIDDEN, 1), lambda i: (0, 0)),
            pl.BlockSpec((1, 1), lambda i: (0, 0)),
        ],
        out_specs=pl.BlockSpec((BM, 1), lambda i: (i, 0)),
        out_shape=jax.ShapeDtypeStruct((ga.shape[1], 1), jnp.float32),
    )(ga, gb, w1, b1, w2, b2, w3, b3)


def _edge_reshape(ei):
    src = ei[0].reshape(NS, EDGE_NB, EDGE_B)
    dst = ei[1].reshape(NS, EDGE_NB, EDGE_B)
    return src, dst


def kernel(x, block0_edge_index, block1_edge_index, block2_edge_index,
           pos_edge_index, neg_edge_index,
           Wself0, Wneigh0, b0, Wself1, Wneigh1, b1, Wself2, Wneigh2, b2,
           Wd1, bd1, Wd2, bd2, Wd3, bd3):
    f32 = jnp.float32
    xc = x.reshape(N_NODES, IN_FEATS // LANE, LANE).transpose(1, 0, 2)
    ws0 = Wself0.reshape(IN_FEATS // LANE, LANE, HIDDEN)
    wn0 = Wneigh0.reshape(IN_FEATS // LANE, LANE, HIDDEN)
    ws1 = Wself1.reshape(HIDDEN // LANE, LANE, HIDDEN)
    wn1 = Wneigh1.reshape(HIDDEN // LANE, LANE, HIDDEN)
    ws2 = Wself2.reshape(HIDDEN // LANE, LANE, HIDDEN)
    wn2 = Wneigh2.reshape(HIDDEN // LANE, LANE, HIDDEN)
    wd1 = Wd1.reshape(HIDDEN // LANE, LANE, HIDDEN)

    zacc = jnp.zeros((PAD_N, LANE), f32)
    ones = jnp.ones((EDGE_B, LANE), f32)

    agg2 = _make_sc_agg(2)
    agg4 = _make_sc_agg(4)

    s0, d0 = _edge_reshape(block0_edge_index)
    s1, d1 = _edge_reshape(block1_edge_index)
    s2, d2 = _edge_reshape(block2_edge_index)

    d0c = block0_edge_index[1].reshape(NC * NS, CNT_NB, EDGE_B)
    d1c = block1_edge_index[1].reshape(NC * NS, CNT_NB, EDGE_B)
    d2c = block2_edge_index[1].reshape(NC * NS, CNT_NB, EDGE_B)
    c0, c1, c2 = _sc_counts(d0c, d1c, d2c, zacc, ones)

    a0 = agg2(xc, s0, d0, zacc)
    h1 = _tc_layer(xc, a0, c0[:, :, :1], ws0, wn0, b0.reshape(1, HIDDEN),
                   relu=True)
    a1 = agg4(h1, s1, d1, zacc)
    h2 = _tc_layer(h1, a1, c1[:, :, :1], ws1, wn1, b1.reshape(1, HIDDEN),
                   relu=True)
    a2 = agg4(h2, s2, d2, zacc)
    h3 = _tc_layer(h2, a2, c2[:, :, :1], ws2, wn2, b2.reshape(1, HIDDEN),
                   relu=False)

    pad_idx = (jnp.arange(PAIR_TOTP - PAIR_TOT, dtype=jnp.int32) * 97
               ) % N_NODES
    qsrc = jnp.concatenate([pos_edge_index[0], neg_edge_index[0], pad_idx])
    qdst = jnp.concatenate([pos_edge_index[1], neg_edge_index[1], pad_idx])
    half = PAIR_TOTP // 2
    nbh = PAIR_NB // 2
    decs = []
    gathered = []
    for lo in (0, half):
        qs = qsrc[lo:lo + half].reshape(NC * NS, nbh, PAIR_B)
        qd = qdst[lo:lo + half].reshape(NC * NS, nbh, PAIR_B)
        gathered.append(_sc_pair_gather(h3, qs, qd, nbh))
    for ga, gb in gathered:
        decs.append(_tc_decoder(ga, gb, wd1, bd1.reshape(1, HIDDEN),
                                Wd2, bd2.reshape(1, HIDDEN),
                                Wd3, bd3.reshape(1, 1)))
    d1, d2 = decs
    h_pos = d1[:N_PAIRS]
    h_neg = jnp.concatenate([d1[N_PAIRS:half], d2[:PAIR_TOT - half]])
    return h_pos, h_neg


# speedup vs baseline: 1.0330x; 1.0330x over previous
import jax
import jax.numpy as jnp
from jax import lax
from jax.experimental import pallas as pl
from jax.experimental.pallas import tpu as pltpu
from jax.experimental.pallas import tpu_sc as plsc

N_NODES = 10000
N_EDGES = 160000
N_PAIRS = 20000
IN_FEATS = 256
HIDDEN = 512

NC = 2
NS = 16
LANE = 128

EDGE_B = 125
EDGE_NB = (N_EDGES // NS) // EDGE_B
EDGE_W = 16

PAD_N = 10240
ROWS_PER_TILE = PAD_N // NS

PAIR_TOT = 2 * N_PAIRS
PAIR_TOTP = 40960
PAIR_PER_TILE = PAIR_TOTP // (NC * NS)
PAIR_B = 128
PAIR_NB = PAIR_PER_TILE // PAIR_B

def _sc_mesh():
    return plsc.VectorSubcoreMesh(
        core_axis_name="c", subcore_axis_name="s",
        num_cores=NC, num_subcores=NS)


def _make_sc_agg(C):
    def body(h_hbm, srcr_hbm, dstr_hbm, zacc_hbm, agg_hbm,
             src_w, dst_w, rows_a, rows_b, acc_sh, gsem_a, gsem_b):
        cid = lax.axis_index("c")
        sid = lax.axis_index("s")
        r0 = sid * ROWS_PER_TILE

        for i in range(C // NC):
            cc = cid * (C // NC) + i
            pltpu.sync_copy(zacc_hbm.at[pl.ds(r0, ROWS_PER_TILE)],
                            acc_sh.at[pl.ds(r0, ROWS_PER_TILE)])
            plsc.subcore_barrier()

            def fire(j, buf, sem):
                pltpu.async_copy(
                    h_hbm.at[cc].at[src_w.at[j]], buf, sem)

            def drain_scat(j, buf, sem):
                pltpu.make_async_copy(
                    h_hbm.at[cc].at[src_w.at[j]], buf, sem).wait()
                pltpu.sync_copy(buf, acc_sh.at[dst_w.at[j]], add=True)

            def win_body(w, carry):
                pltpu.sync_copy(
                    srcr_hbm.at[sid, pl.ds(w * EDGE_W, EDGE_W)], src_w)
                pltpu.sync_copy(
                    dstr_hbm.at[sid, pl.ds(w * EDGE_W, EDGE_W)], dst_w)
                fire(0, rows_a, gsem_a)

                def pair_body(t, c2):
                    j0 = 2 * t
                    fire(j0 + 1, rows_b, gsem_b)
                    drain_scat(j0, rows_a, gsem_a)

                    @pl.when(j0 + 2 < EDGE_W)
                    def _():
                        fire(j0 + 2, rows_a, gsem_a)
                    drain_scat(j0 + 1, rows_b, gsem_b)
                    return c2
                lax.fori_loop(0, EDGE_W // 2, pair_body, 0)
                return carry
            lax.fori_loop(0, EDGE_NB // EDGE_W, win_body, 0)
            plsc.subcore_barrier()

            pltpu.sync_copy(acc_sh.at[pl.ds(r0, ROWS_PER_TILE)],
                            agg_hbm.at[cc, pl.ds(r0, ROWS_PER_TILE)])
            plsc.subcore_barrier()

    return pl.kernel(
        body,
        out_type=jax.ShapeDtypeStruct((C, PAD_N, LANE), jnp.float32),
        mesh=_sc_mesh(),
        scratch_types=[
            pltpu.VMEM((EDGE_W, EDGE_B), jnp.int32),
            pltpu.VMEM((EDGE_W, EDGE_B), jnp.int32),
            pltpu.VMEM((EDGE_B, LANE), jnp.float32),
            pltpu.VMEM((EDGE_B, LANE), jnp.float32),
            pltpu.VMEM_SHARED((PAD_N, LANE), jnp.float32),
            pltpu.SemaphoreType.DMA,
            pltpu.SemaphoreType.DMA,
        ],
    )


CNT_NB = N_EDGES // (NC * NS) // EDGE_B


def _sc_counts(dr, zacc, ones):
    def body(d_hbm, zacc_hbm, ones_hbm, c_hbm, dst_v, ones_v, cnt_sh, csem):
        cid = lax.axis_index("c")
        sid = lax.axis_index("s")
        wid = cid * NS + sid
        r0 = sid * ROWS_PER_TILE
        pltpu.sync_copy(ones_hbm, ones_v)
        pltpu.sync_copy(d_hbm.at[wid], dst_v)
        pltpu.sync_copy(zacc_hbm.at[pl.ds(r0, ROWS_PER_TILE)],
                        cnt_sh.at[pl.ds(r0, ROWS_PER_TILE)])
        plsc.subcore_barrier()

        def cnt_body(j, carry):
            pltpu.async_copy(ones_v, cnt_sh.at[dst_v.at[j]],
                             csem, add=True)
            return carry
        lax.fori_loop(0, CNT_NB, cnt_body, 0)

        def cnt_drain(j, carry):
            pltpu.make_async_copy(ones_v, cnt_sh.at[dst_v.at[j]],
                                  csem).wait()
            return carry
        lax.fori_loop(0, CNT_NB, cnt_drain, 0)
        plsc.subcore_barrier()
        pltpu.sync_copy(cnt_sh.at[pl.ds(r0, ROWS_PER_TILE)],
                        c_hbm.at[cid, pl.ds(r0, ROWS_PER_TILE)])
        plsc.subcore_barrier()

    k = pl.kernel(
        body,
        out_type=jax.ShapeDtypeStruct((NC, PAD_N, LANE), jnp.float32),
        mesh=_sc_mesh(),
        scratch_types=[
            pltpu.VMEM((CNT_NB, EDGE_B), jnp.int32),
            pltpu.VMEM((EDGE_B, LANE), jnp.float32),
            pltpu.VMEM_SHARED((PAD_N, LANE), jnp.float32),
            pltpu.SemaphoreType.DMA,
        ],
    )
    return k(dr, zacc, ones)


def _sc_pair_gather(h, qsrc, qdst, nb):
    per_tile = nb * PAIR_B
    tot = (NC * NS) * per_tile

    def body(h_hbm, qs_hbm, qd_hbm, ga_hbm, gb_hbm,
             qs_v, qd_v, buf_a, buf_b, sem_a, sem_b):
        cid = lax.axis_index("c")
        sid = lax.axis_index("s")
        wid = cid * NS + sid
        base = wid * per_tile
        pltpu.sync_copy(qs_hbm.at[wid], qs_v)
        pltpu.sync_copy(qd_hbm.at[wid], qd_v)
        for c in range(4):
            def pair_body(j, carry):
                cpa = pltpu.async_copy(
                    h_hbm.at[c].at[qs_v.at[j]], buf_a, sem_a)
                cpb = pltpu.async_copy(
                    h_hbm.at[c].at[qd_v.at[j]], buf_b, sem_b)
                cpa.wait()
                pltpu.sync_copy(buf_a,
                                ga_hbm.at[c, pl.ds(base + j * PAIR_B, PAIR_B)])
                cpb.wait()
                pltpu.sync_copy(buf_b,
                                gb_hbm.at[c, pl.ds(base + j * PAIR_B, PAIR_B)])
                return carry
            lax.fori_loop(0, nb, pair_body, 0)

    k = pl.kernel(
        body,
        out_type=(
            jax.ShapeDtypeStruct((4, tot, LANE), jnp.float32),
            jax.ShapeDtypeStruct((4, tot, LANE), jnp.float32),
        ),
        mesh=_sc_mesh(),
        scratch_types=[
            pltpu.VMEM((nb, PAIR_B), jnp.int32),
            pltpu.VMEM((nb, PAIR_B), jnp.int32),
            pltpu.VMEM((PAIR_B, LANE), jnp.float32),
            pltpu.VMEM((PAIR_B, LANE), jnp.float32),
            pltpu.SemaphoreType.DMA,
            pltpu.SemaphoreType.DMA,
        ],
    )
    return k(h, qsrc, qdst)


def _tc_layer(h, agg, cnt, ws, wn, b, relu):
    C = h.shape[0]
    BM = 1000
    grid = (N_NODES // BM,)

    def body(h_ref, agg_ref, cnt_ref, ws_ref, wn_ref, b_ref, out_ref):
        cnt = cnt_ref[0] + cnt_ref[1]
        inv = 1.0 / jnp.maximum(cnt, 1.0)
        s = jnp.zeros((BM, HIDDEN), jnp.float32)
        for c in range(C):
            s += jnp.dot(h_ref[c], ws_ref[c],
                         preferred_element_type=jnp.float32)
            s += jnp.dot(agg_ref[c] * inv, wn_ref[c],
                         preferred_element_type=jnp.float32)
        s += b_ref[...]
        if relu:
            s = jnp.maximum(s, 0.0)
        for c2 in range(HIDDEN // LANE):
            out_ref[c2] = s[:, c2 * LANE:(c2 + 1) * LANE]

    return pl.pallas_call(
        body,
        grid=grid,
        in_specs=[
            pl.BlockSpec((C, BM, LANE), lambda i: (0, i, 0)),
            pl.BlockSpec((C, BM, LANE), lambda i: (0, i, 0)),
            pl.BlockSpec((NC, BM, 1), lambda i: (0, i, 0)),
            pl.BlockSpec((C, LANE, HIDDEN), lambda i: (0, 0, 0)),
            pl.BlockSpec((C, LANE, HIDDEN), lambda i: (0, 0, 0)),
            pl.BlockSpec((1, HIDDEN), lambda i: (0, 0)),
        ],
        out_specs=pl.BlockSpec((HIDDEN // LANE, BM, LANE), lambda i: (0, i, 0)),
        out_shape=jax.ShapeDtypeStruct((HIDDEN // LANE, N_NODES, LANE),
                                       jnp.float32),
    )(h, agg, cnt, ws, wn, b)


def _tc_decoder(ga, gb, w1, b1, w2, b2, w3, b3):
    BM = 1024
    grid = (ga.shape[1] // BM,)

    def body(ga_ref, gb_ref, w1_ref, b1_ref, w2_ref, b2_ref, w3_ref, b3_ref,
             out_ref):
        t = jnp.zeros((BM, HIDDEN), jnp.float32)
        for c in range(4):
            e = ga_ref[c] * gb_ref[c]
            t += jnp.dot(e, w1_ref[c], preferred_element_type=jnp.float32)
        t = jnp.maximum(t + b1_ref[...], 0.0)
        t = jnp.maximum(
            jnp.dot(t, w2_ref[...], preferred_element_type=jnp.float32)
            + b2_ref[...], 0.0)
        out_ref[...] = (
            jnp.dot(t, w3_ref[...], preferred_element_type=jnp.float32)
            + b3_ref[...])

    return pl.pallas_call(
        body,
        grid=grid,
        in_specs=[
            pl.BlockSpec((4, BM, LANE), lambda i: (0, i, 0)),
            pl.BlockSpec((4, BM, LANE), lambda i: (0, i, 0)),
            pl.BlockSpec((4, LANE, HIDDEN), lambda i: (0, 0, 0)),
            pl.BlockSpec((1, HIDDEN), lambda i: (0, 0)),
            pl.BlockSpec((HIDDEN, HIDDEN), lambda i: (0, 0)),
            pl.BlockSpec((1, HIDDEN), lambda i: (0, 0)),
            pl.BlockSpec((HIDDEN, 1), lambda i: (0, 0)),
            pl.BlockSpec((1, 1), lambda i: (0, 0)),
        ],
        out_specs=pl.BlockSpec((BM, 1), lambda i: (i, 0)),
        out_shape=jax.ShapeDtypeStruct((ga.shape[1], 1), jnp.float32),
    )(ga, gb, w1, b1, w2, b2, w3, b3)


def _edge_reshape(ei):
    src = ei[0].reshape(NS, EDGE_NB, EDGE_B)
    dst = ei[1].reshape(NS, EDGE_NB, EDGE_B)
    return src, dst


def kernel(x, block0_edge_index, block1_edge_index, block2_edge_index,
           pos_edge_index, neg_edge_index,
           Wself0, Wneigh0, b0, Wself1, Wneigh1, b1, Wself2, Wneigh2, b2,
           Wd1, bd1, Wd2, bd2, Wd3, bd3):
    f32 = jnp.float32
    xc = x.reshape(N_NODES, IN_FEATS // LANE, LANE).transpose(1, 0, 2)
    ws0 = Wself0.reshape(IN_FEATS // LANE, LANE, HIDDEN)
    wn0 = Wneigh0.reshape(IN_FEATS // LANE, LANE, HIDDEN)
    ws1 = Wself1.reshape(HIDDEN // LANE, LANE, HIDDEN)
    wn1 = Wneigh1.reshape(HIDDEN // LANE, LANE, HIDDEN)
    ws2 = Wself2.reshape(HIDDEN // LANE, LANE, HIDDEN)
    wn2 = Wneigh2.reshape(HIDDEN // LANE, LANE, HIDDEN)
    wd1 = Wd1.reshape(HIDDEN // LANE, LANE, HIDDEN)

    zacc = jnp.zeros((PAD_N, LANE), f32)
    ones = jnp.ones((EDGE_B, LANE), f32)

    agg2 = _make_sc_agg(2)
    agg4 = _make_sc_agg(4)

    s0, d0 = _edge_reshape(block0_edge_index)
    s1, d1 = _edge_reshape(block1_edge_index)
    s2, d2 = _edge_reshape(block2_edge_index)

    d0c = block0_edge_index[1].reshape(NC * NS, CNT_NB, EDGE_B)
    d1c = block1_edge_index[1].reshape(NC * NS, CNT_NB, EDGE_B)
    d2c = block2_edge_index[1].reshape(NC * NS, CNT_NB, EDGE_B)
    c0 = _sc_counts(d0c, zacc, ones)
    c1 = _sc_counts(d1c, zacc, ones)
    c2 = _sc_counts(d2c, zacc, ones)

    a0 = agg2(xc, s0, d0, zacc)
    h1 = _tc_layer(xc, a0, c0[:, :, :1], ws0, wn0, b0.reshape(1, HIDDEN),
                   relu=True)
    a1 = agg4(h1, s1, d1, zacc)
    h2 = _tc_layer(h1, a1, c1[:, :, :1], ws1, wn1, b1.reshape(1, HIDDEN),
                   relu=True)
    a2 = agg4(h2, s2, d2, zacc)
    h3 = _tc_layer(h2, a2, c2[:, :, :1], ws2, wn2, b2.reshape(1, HIDDEN),
                   relu=False)

    pad_idx = (jnp.arange(PAIR_TOTP - PAIR_TOT, dtype=jnp.int32) * 97
               ) % N_NODES
    qsrc = jnp.concatenate([pos_edge_index[0], neg_edge_index[0], pad_idx])
    qdst = jnp.concatenate([pos_edge_index[1], neg_edge_index[1], pad_idx])
    half = PAIR_TOTP // 2
    nbh = PAIR_NB // 2
    decs = []
    gathered = []
    for lo in (0, half):
        qs = qsrc[lo:lo + half].reshape(NC * NS, nbh, PAIR_B)
        qd = qdst[lo:lo + half].reshape(NC * NS, nbh, PAIR_B)
        gathered.append(_sc_pair_gather(h3, qs, qd, nbh))
    for ga, gb in gathered:
        decs.append(_tc_decoder(ga, gb, wd1, bd1.reshape(1, HIDDEN),
                                Wd2, bd2.reshape(1, HIDDEN),
                                Wd3, bd3.reshape(1, 1)))
    d1, d2 = decs
    h_pos = d1[:N_PAIRS]
    h_neg = jnp.concatenate([d1[N_PAIRS:half], d2[:PAIR_TOT - half]])
    return h_pos, h_neg
